# Initial kernel scaffold; baseline (speedup 1.0000x reference)
import jax
import jax.numpy as jnp
from jax.experimental import pallas as pl


def kernel(x, edge_index, edge_weight, W, bias):
    raise NotImplementedError("write your pallas kernel here")


# speedup vs baseline: 21.6243x; 21.6243x over previous
import functools

import jax
import jax.numpy as jnp
from jax import lax
from jax.experimental import pallas as pl
from jax.experimental.pallas import tpu as pltpu
import jax.experimental.pallas.tpu_sc as plsc

N = 10000
C = 128
NC, NS, L = 2, 16, 16
NW = NC * NS
NPAD = 10240
SLICE = NPAD // NS
B = 128
ROWS_W = 80
ROWS = NW * ROWS_W
EPAD = ROWS * B
HROWS = ROWS_W // 2
NSL = N // NS

_mesh = plsc.VectorSubcoreMesh(
    core_axis_name="c", subcore_axis_name="s", num_cores=NC, num_subcores=NS)


@functools.partial(
    pl.kernel,
    out_type=jax.ShapeDtypeStruct((NC, NPAD), jnp.float32),
    mesh=_mesh,
    compiler_params=pltpu.CompilerParams(needs_layout_passes=False),
    scratch_types=[
        pltpu.VMEM((ROWS_W, B), jnp.int32),
        pltpu.VMEM((ROWS_W, B), jnp.float32),
        pltpu.VMEM((SLICE,), jnp.float32),
        pltpu.VMEM_SHARED((NPAD,), jnp.float32),
    ],
)
def _deg_kernel(col_hbm, ew_hbm, pdeg_hbm, colv, ewv, zv, sdeg):
    c = lax.axis_index("c")
    s = lax.axis_index("s")
    for k in range(SLICE // L):
        zv[pl.ds(k * L, L)] = jnp.zeros((L,), jnp.float32)
    pltpu.sync_copy(zv, sdeg.at[pl.ds(s * SLICE, SLICE)])
    plsc.subcore_barrier()
    base = (c * NS + s) * ROWS_W
    pltpu.sync_copy(col_hbm.at[pl.ds(base, ROWS_W)], colv)
    pltpu.sync_copy(ew_hbm.at[pl.ds(base, ROWS_W)], ewv)

    def step(j, carry):
        pltpu.sync_copy(ewv.at[j], sdeg.at[colv.at[j]], add=True)
        return carry

    lax.fori_loop(0, ROWS_W, step, 0)
    plsc.subcore_barrier()
    pltpu.sync_copy(sdeg.at[pl.ds(s * SLICE, SLICE)],
                    pdeg_hbm.at[c, pl.ds(s * SLICE, SLICE)])


@functools.partial(
    pl.kernel,
    out_type=(jax.ShapeDtypeStruct((NC, NPAD, C), jnp.float32),
              jax.ShapeDtypeStruct((NPAD,), jnp.float32)),
    mesh=_mesh,
    compiler_params=pltpu.CompilerParams(needs_layout_passes=False),
    scratch_types=[
        pltpu.VMEM((HROWS, B), jnp.int32),
        pltpu.VMEM((HROWS, B), jnp.int32),
        pltpu.VMEM((HROWS, B), jnp.float32),
        pltpu.VMEM((NPAD,), jnp.float32),
        pltpu.VMEM((SLICE,), jnp.float32),
        pltpu.VMEM((SLICE,), jnp.float32),
        pltpu.VMEM((SLICE,), jnp.float32),
        pltpu.VMEM((B,), jnp.float32),
        pltpu.VMEM((B, C), jnp.float32),
        pltpu.VMEM_SHARED((NPAD, C), jnp.float32),
        pltpu.VMEM_SHARED((NPAD,), jnp.float32),
    ],
)
def _agg_kernel(row_hbm, col_hbm, ew_hbm, x_hbm, pdeg_hbm,
                pagg_hbm, dis_hbm,
                rowv, colv, eww, dis, pa, pb, dloc, scale, msg, sagg, sdis):
    c = lax.axis_index("c")
    s = lax.axis_index("s")

    def zrow(j, carry):
        for k in range(C // L):
            msg[j, pl.ds(k * L, L)] = jnp.zeros((L,), jnp.float32)
        return carry

    lax.fori_loop(0, B, zrow, 0)
    for m in range(SLICE // B):
        pltpu.sync_copy(msg, sagg.at[pl.ds(s * SLICE + m * B, B)])

    off = s * SLICE
    pltpu.sync_copy(pdeg_hbm.at[0, pl.ds(off, SLICE)], pa)
    pltpu.sync_copy(pdeg_hbm.at[1, pl.ds(off, SLICE)], pb)
    def newt(k, carry):
        for h in range(2):
            sl = pl.ds(k * 2 * L + h * L, L)
            d = pa[sl] + pb[sl] + 2.0
            y = jnp.full((L,), 0.0015, jnp.float32)
            for _ in range(26):
                y = y * (1.5 - 0.5 * d * y * y)
            dloc[sl] = jnp.where(d > 0.0, y, 0.0)
        return carry

    lax.fori_loop(0, SLICE // (2 * L), newt, 0)
    pltpu.sync_copy(dloc, sdis.at[pl.ds(off, SLICE)])
    plsc.subcore_barrier()
    pltpu.sync_copy(sdis, dis)

    @pl.when(jnp.logical_and(c == 0, s == 0))
    def _():
        pltpu.sync_copy(sdis, dis_hbm)

    base = (c * NS + s) * ROWS_W

    def batch(j, carry):
        pltpu.sync_copy(x_hbm.at[rowv.at[j]], msg)
        for k in range(B // L):
            rv = rowv[j, pl.ds(k * L, L)]
            dv = plsc.load_gather(dis, [rv])
            scale[pl.ds(k * L, L)] = eww[j, pl.ds(k * L, L)] * dv

        def estep(e, cc):
            w = plsc.load_gather(scale, [jnp.zeros((L,), jnp.int32) + e])
            for k in range(C // L):
                msg[e, pl.ds(k * L, L)] = msg[e, pl.ds(k * L, L)] * w
            return cc

        lax.fori_loop(0, B, estep, 0)
        pltpu.sync_copy(msg, sagg.at[colv.at[j]], add=True)
        return carry

    for h in range(ROWS_W // HROWS):
        hb = base + h * HROWS
        pltpu.sync_copy(row_hbm.at[pl.ds(hb, HROWS)], rowv)
        pltpu.sync_copy(col_hbm.at[pl.ds(hb, HROWS)], colv)
        pltpu.sync_copy(ew_hbm.at[pl.ds(hb, HROWS)], eww)
        lax.fori_loop(0, HROWS, batch, 0)

    plsc.subcore_barrier()
    pltpu.sync_copy(sagg.at[pl.ds(s * SLICE, SLICE)],
                    pagg_hbm.at[c, pl.ds(s * SLICE, SLICE)])


BLK = 2000


def _fin_body(dis_ref, pagg_ref, x_ref, wt_ref, b_ref, out_ref):
    dis = dis_ref[...]
    agg = pagg_ref[0] + pagg_ref[1]
    y = dis * agg + (2.0 * dis * dis) * x_ref[...]
    out_ref[...] = (
        jnp.dot(y, wt_ref[...], preferred_element_type=jnp.float32)
        + b_ref[...])


def kernel(x, edge_index, edge_weight, W, bias):
    row = edge_index[0]
    col = edge_index[1]
    e = edge_weight.shape[0]
    pad = EPAD - e
    pad_idx = jnp.arange(pad, dtype=jnp.int32) % N
    rowp = jnp.concatenate([row, pad_idx]).reshape(ROWS, B)
    colp = jnp.concatenate([col, pad_idx]).reshape(ROWS, B)
    ewp = jnp.concatenate(
        [edge_weight, jnp.zeros((pad,), jnp.float32)]).reshape(ROWS, B)

    pdeg = _deg_kernel(colp, ewp)
    pagg, dis = _agg_kernel(rowp, colp, ewp, x, pdeg)

    out = pl.pallas_call(
        _fin_body,
        grid=(N // BLK,),
        in_specs=[
            pl.BlockSpec((BLK, 1), lambda i: (i, 0)),
            pl.BlockSpec((NC, BLK, C), lambda i: (0, i, 0)),
            pl.BlockSpec((BLK, C), lambda i: (i, 0)),
            pl.BlockSpec((C, C), lambda i: (0, 0)),
            pl.BlockSpec((1, C), lambda i: (0, 0)),
        ],
        out_specs=pl.BlockSpec((BLK, C), lambda i: (i, 0)),
        out_shape=jax.ShapeDtypeStruct((N, C), jnp.float32),
    )(dis.reshape(NPAD, 1), pagg, x, W.T, bias.reshape(1, C))
    return (out, out)


# speedup vs baseline: 28.2939x; 1.3084x over previous
import functools

import jax
import jax.numpy as jnp
from jax import lax
from jax.experimental import pallas as pl
from jax.experimental.pallas import tpu as pltpu
import jax.experimental.pallas.tpu_sc as plsc

N = 10000
C = 128
NC, NS, L = 2, 16, 16
NW = NC * NS
NPAD = 10240
SLICE = NPAD // NS
B = 128
ROWS_W = 80
ROWS = NW * ROWS_W
EPAD = ROWS * B
HROWS = ROWS_W // 2
NSL = N // NS

_mesh = plsc.VectorSubcoreMesh(
    core_axis_name="c", subcore_axis_name="s", num_cores=NC, num_subcores=NS)


@functools.partial(
    pl.kernel,
    out_type=jax.ShapeDtypeStruct((NC, NPAD), jnp.float32),
    mesh=_mesh,
    compiler_params=pltpu.CompilerParams(needs_layout_passes=False),
    scratch_types=[
        pltpu.VMEM((ROWS_W, B), jnp.int32),
        pltpu.VMEM((ROWS_W, B), jnp.float32),
        pltpu.VMEM((SLICE,), jnp.float32),
        pltpu.VMEM_SHARED((NPAD,), jnp.float32),
    ],
)
def _deg_kernel(col_hbm, ew_hbm, pdeg_hbm, colv, ewv, zv, sdeg):
    c = lax.axis_index("c")
    s = lax.axis_index("s")
    for k in range(SLICE // L):
        zv[pl.ds(k * L, L)] = jnp.zeros((L,), jnp.float32)
    pltpu.sync_copy(zv, sdeg.at[pl.ds(s * SLICE, SLICE)])
    plsc.subcore_barrier()
    base = (c * NS + s) * ROWS_W
    pltpu.sync_copy(col_hbm.at[pl.ds(base, ROWS_W)], colv)
    pltpu.sync_copy(ew_hbm.at[pl.ds(base, ROWS_W)], ewv)

    def step(j, carry):
        pltpu.sync_copy(ewv.at[j], sdeg.at[colv.at[j]], add=True)
        return carry

    lax.fori_loop(0, ROWS_W, step, 0)
    plsc.subcore_barrier()
    pltpu.sync_copy(sdeg.at[pl.ds(s * SLICE, SLICE)],
                    pdeg_hbm.at[c, pl.ds(s * SLICE, SLICE)])


CH = 8
NCHUNK = ROWS_W // CH
GBYTES = B * C * 4


@functools.partial(
    pl.kernel,
    out_type=(jax.ShapeDtypeStruct((NC, NPAD, C), jnp.float32),
              jax.ShapeDtypeStruct((NPAD,), jnp.float32)),
    mesh=_mesh,
    compiler_params=pltpu.CompilerParams(needs_layout_passes=False),
    scratch_types=[
        pltpu.VMEM((CH, B), jnp.int32),
        pltpu.VMEM((CH, B), jnp.int32),
        pltpu.VMEM((CH, B), jnp.float32),
        pltpu.VMEM((SLICE,), jnp.float32),
        pltpu.VMEM((SLICE,), jnp.float32),
        pltpu.VMEM((NPAD,), jnp.float32),
        pltpu.VMEM((B,), jnp.float32),
        pltpu.VMEM((B, C), jnp.float32),
        pltpu.VMEM((B, C), jnp.float32),
        pltpu.VMEM_SHARED((NPAD, C), jnp.float32),
        pltpu.VMEM_SHARED((NPAD,), jnp.float32),
        pltpu.SemaphoreType.DMA,
        pltpu.SemaphoreType.DMA,
        pltpu.SemaphoreType.DMA,
        pltpu.SemaphoreType.DMA,
    ],
)
def _agg_kernel(row_hbm, col_hbm, ew_hbm, x_hbm, pdeg_hbm,
                pagg_hbm, dis_hbm,
                rowstg, colstg, ewstg, pa, pb, dis, scale, msg0, msg1,
                sagg, sdis, semg0, semg1, sems0, sems1):
    c = lax.axis_index("c")
    s = lax.axis_index("s")

    def zrow(j, carry):
        for k in range(C // L):
            msg0[j, pl.ds(k * L, L)] = jnp.zeros((L,), jnp.float32)
        return carry

    lax.fori_loop(0, B, zrow, 0)
    for m in range(SLICE // B):
        pltpu.sync_copy(msg0, sagg.at[pl.ds(s * SLICE + m * B, B)])

    off = s * SLICE
    pltpu.sync_copy(pdeg_hbm.at[0, pl.ds(off, SLICE)], pa)
    pltpu.sync_copy(pdeg_hbm.at[1, pl.ds(off, SLICE)], pb)

    def newt(k, carry):
        sl = pl.ds(k * L, L)
        d = pa[sl] + pb[sl] + 2.0
        y = jnp.full((L,), 0.0015, jnp.float32)
        for _ in range(26):
            y = y * (1.5 - 0.5 * d * y * y)
        dis[pl.ds(off + k * L, L)] = jnp.where(d > 0.0, y, 0.0)
        return carry

    lax.fori_loop(0, SLICE // L, newt, 0)
    pltpu.sync_copy(dis.at[pl.ds(off, SLICE)], sdis.at[pl.ds(off, SLICE)])
    plsc.subcore_barrier()
    pltpu.sync_copy(sdis, dis)

    @pl.when(jnp.logical_and(c == 0, s == 0))
    def _():
        pltpu.sync_copy(sdis, dis_hbm)

    base = (c * NS + s) * ROWS_W

    def _scale(j):
        for k in range(B // L):
            rv = rowstg[j, pl.ds(k * L, L)]
            dv = plsc.load_gather(dis, [rv])
            scale[pl.ds(k * L, L)] = ewstg[j, pl.ds(k * L, L)] * dv

    def _apply(mref):
        def ebody(t, cc):
            for u in range(4):
                e = t * 4 + u
                w = plsc.load_gather(scale, [jnp.zeros((L,), jnp.int32) + e])
                for k in range(C // L):
                    mref[e, pl.ds(k * L, L)] = mref[e, pl.ds(k * L, L)] * w
            return cc

        lax.fori_loop(0, B // 4, ebody, 0)

    def chunk(ch, carry):
        @pl.when(ch > 0)
        def _():
            pltpu.make_async_copy(msg0, sagg.at[colstg.at[0]], sems0).wait()
            pltpu.make_async_copy(msg1, sagg.at[colstg.at[1]], sems1).wait()

        hb = base + ch * CH
        pltpu.sync_copy(row_hbm.at[pl.ds(hb, CH)], rowstg)
        pltpu.sync_copy(col_hbm.at[pl.ds(hb, CH)], colstg)
        pltpu.sync_copy(ew_hbm.at[pl.ds(hb, CH)], ewstg)
        pltpu.async_copy(x_hbm.at[rowstg.at[0]], msg0, semg0)

        def pair(t, cc):
            j0 = 2 * t
            j1 = j0 + 1
            pltpu.make_async_copy(x_hbm.at[rowstg.at[j0]], msg0, semg0).wait()

            @pl.when(t > 0)
            def _():
                pltpu.make_async_copy(
                    msg1, sagg.at[colstg.at[j1]], sems1).wait()

            pltpu.async_copy(x_hbm.at[rowstg.at[j1]], msg1, semg1)
            _scale(j0)
            _apply(msg0)
            pltpu.async_copy(msg0, sagg.at[colstg.at[j0]], sems0, add=True)
            pltpu.make_async_copy(x_hbm.at[rowstg.at[j1]], msg1, semg1).wait()
            _scale(j1)
            _apply(msg1)

            @pl.when(t < CH // 2 - 1)
            def _():
                pltpu.make_async_copy(
                    msg0, sagg.at[colstg.at[j0]], sems0).wait()
                pltpu.async_copy(x_hbm.at[rowstg.at[j0 + 2]], msg0, semg0)

            pltpu.async_copy(msg1, sagg.at[colstg.at[j1]], sems1, add=True)
            return cc

        lax.fori_loop(0, CH // 2, pair, 0)
        return carry

    lax.fori_loop(0, NCHUNK, chunk, 0)
    pltpu.make_async_copy(msg0, sagg.at[colstg.at[0]], sems0).wait()
    pltpu.make_async_copy(msg1, sagg.at[colstg.at[1]], sems1).wait()
    plsc.subcore_barrier()
    pltpu.sync_copy(sagg.at[pl.ds(s * SLICE, SLICE)],
                    pagg_hbm.at[c, pl.ds(s * SLICE, SLICE)])


BLK = 2000


def _fin_body(dis_ref, pagg_ref, x_ref, wt_ref, b_ref, out_ref):
    dis = dis_ref[...]
    agg = pagg_ref[0] + pagg_ref[1]
    y = dis * agg + (2.0 * dis * dis) * x_ref[...]
    out_ref[...] = (
        jnp.dot(y, wt_ref[...], preferred_element_type=jnp.float32)
        + b_ref[...])


def kernel(x, edge_index, edge_weight, W, bias):
    row = edge_index[0]
    col = edge_index[1]
    e = edge_weight.shape[0]
    pad = EPAD - e
    pad_idx = jnp.arange(pad, dtype=jnp.int32) % N
    rowp = jnp.concatenate([row, pad_idx]).reshape(ROWS, B)
    colp = jnp.concatenate([col, pad_idx]).reshape(ROWS, B)
    ewp = jnp.concatenate(
        [edge_weight, jnp.zeros((pad,), jnp.float32)]).reshape(ROWS, B)

    pdeg = _deg_kernel(colp, ewp)
    pagg, dis = _agg_kernel(rowp, colp, ewp, x, pdeg)

    out = pl.pallas_call(
        _fin_body,
        grid=(N // BLK,),
        in_specs=[
            pl.BlockSpec((BLK, 1), lambda i: (i, 0)),
            pl.BlockSpec((NC, BLK, C), lambda i: (0, i, 0)),
            pl.BlockSpec((BLK, C), lambda i: (i, 0)),
            pl.BlockSpec((C, C), lambda i: (0, 0)),
            pl.BlockSpec((1, C), lambda i: (0, 0)),
        ],
        out_specs=pl.BlockSpec((BLK, C), lambda i: (i, 0)),
        out_shape=jax.ShapeDtypeStruct((N, C), jnp.float32),
    )(dis.reshape(NPAD, 1), pagg, x, W.T, bias.reshape(1, C))
    return (out, out)


# speedup vs baseline: 28.8592x; 1.0200x over previous
import functools

import jax
import jax.numpy as jnp
from jax import lax
from jax.experimental import pallas as pl
from jax.experimental.pallas import tpu as pltpu
import jax.experimental.pallas.tpu_sc as plsc

N = 10000
C = 128
NC, NS, L = 2, 16, 16
NW = NC * NS
NPAD = 10240
SLICE = NPAD // NS
B = 128
ROWS_W = 80
ROWS = NW * ROWS_W
EPAD = ROWS * B
HROWS = ROWS_W // 2
NSL = N // NS

_mesh = plsc.VectorSubcoreMesh(
    core_axis_name="c", subcore_axis_name="s", num_cores=NC, num_subcores=NS)


@functools.partial(
    pl.kernel,
    out_type=jax.ShapeDtypeStruct((NC, NPAD), jnp.float32),
    mesh=_mesh,
    compiler_params=pltpu.CompilerParams(needs_layout_passes=False),
    scratch_types=[
        pltpu.VMEM((ROWS_W, B), jnp.int32),
        pltpu.VMEM((ROWS_W, B), jnp.float32),
        pltpu.VMEM((SLICE,), jnp.float32),
        pltpu.VMEM_SHARED((NPAD,), jnp.float32),
        pltpu.SemaphoreType.DMA,
    ],
)
def _deg_kernel(col_hbm, ew_hbm, pdeg_hbm, colv, ewv, zv, sdeg, sem):
    c = lax.axis_index("c")
    s = lax.axis_index("s")
    for k in range(SLICE // L):
        zv[pl.ds(k * L, L)] = jnp.zeros((L,), jnp.float32)
    pltpu.sync_copy(zv, sdeg.at[pl.ds(s * SLICE, SLICE)])
    plsc.subcore_barrier()
    base = (c * NS + s) * ROWS_W
    pltpu.sync_copy(col_hbm.at[pl.ds(base, ROWS_W)], colv)
    pltpu.sync_copy(ew_hbm.at[pl.ds(base, ROWS_W)], ewv)

    def step(j, carry):
        pltpu.async_copy(ewv.at[j], sdeg.at[colv.at[j]], sem, add=True)
        return carry

    lax.fori_loop(0, ROWS_W, step, 0)

    def drain(j, carry):
        pltpu.make_async_copy(ewv.at[0], sdeg.at[colv.at[0]], sem).wait()
        return carry

    lax.fori_loop(0, ROWS_W, drain, 0)
    plsc.subcore_barrier()
    pltpu.sync_copy(sdeg.at[pl.ds(s * SLICE, SLICE)],
                    pdeg_hbm.at[c, pl.ds(s * SLICE, SLICE)])


CH = 8
NCHUNK = ROWS_W // CH
GBYTES = B * C * 4


@functools.partial(
    pl.kernel,
    out_type=(jax.ShapeDtypeStruct((NC, NPAD, C), jnp.float32),
              jax.ShapeDtypeStruct((NPAD,), jnp.float32)),
    mesh=_mesh,
    compiler_params=pltpu.CompilerParams(needs_layout_passes=False),
    scratch_types=[
        pltpu.VMEM((CH, B), jnp.int32),
        pltpu.VMEM((CH, B), jnp.int32),
        pltpu.VMEM((CH, B), jnp.float32),
        pltpu.VMEM((SLICE,), jnp.float32),
        pltpu.VMEM((SLICE,), jnp.float32),
        pltpu.VMEM((NPAD,), jnp.float32),
        pltpu.VMEM((B,), jnp.float32),
        pltpu.VMEM((B, C), jnp.float32),
        pltpu.VMEM((B, C), jnp.float32),
        pltpu.VMEM_SHARED((NPAD, C), jnp.float32),
        pltpu.VMEM_SHARED((NPAD,), jnp.float32),
        pltpu.SemaphoreType.DMA,
        pltpu.SemaphoreType.DMA,
        pltpu.SemaphoreType.DMA,
        pltpu.SemaphoreType.DMA,
    ],
)
def _agg_kernel(row_hbm, col_hbm, ew_hbm, x_hbm, pdeg_hbm,
                pagg_hbm, dis_hbm,
                rowstg, colstg, ewstg, pa, pb, dis, scale, msg0, msg1,
                sagg, sdis, semg0, semg1, sems0, sems1):
    c = lax.axis_index("c")
    s = lax.axis_index("s")

    def zrow(j, carry):
        for k in range(C // L):
            msg0[j, pl.ds(k * L, L)] = jnp.zeros((L,), jnp.float32)
        return carry

    lax.fori_loop(0, B, zrow, 0)
    for m in range(SLICE // B):
        pltpu.sync_copy(msg0, sagg.at[pl.ds(s * SLICE + m * B, B)])

    off = s * SLICE
    pltpu.sync_copy(pdeg_hbm.at[0, pl.ds(off, SLICE)], pa)
    pltpu.sync_copy(pdeg_hbm.at[1, pl.ds(off, SLICE)], pb)

    def newt(k, carry):
        sl = pl.ds(k * L, L)
        d = pa[sl] + pb[sl] + 2.0
        y = jnp.full((L,), 0.0015, jnp.float32)
        for _ in range(26):
            y = y * (1.5 - 0.5 * d * y * y)
        dis[pl.ds(off + k * L, L)] = jnp.where(d > 0.0, y, 0.0)
        return carry

    lax.fori_loop(0, SLICE // L, newt, 0)
    pltpu.sync_copy(dis.at[pl.ds(off, SLICE)], sdis.at[pl.ds(off, SLICE)])
    plsc.subcore_barrier()
    pltpu.sync_copy(sdis, dis)

    @pl.when(jnp.logical_and(c == 0, s == 0))
    def _():
        pltpu.sync_copy(sdis, dis_hbm)

    base = (c * NS + s) * ROWS_W

    def _scale(j):
        for k in range(B // L):
            rv = rowstg[j, pl.ds(k * L, L)]
            dv = plsc.load_gather(dis, [rv])
            scale[pl.ds(k * L, L)] = ewstg[j, pl.ds(k * L, L)] * dv

    def _apply(mref):
        def ebody(t, cc):
            for u in range(4):
                e = t * 4 + u
                w = plsc.load_gather(scale, [jnp.zeros((L,), jnp.int32) + e])
                for k in range(C // L):
                    mref[e, pl.ds(k * L, L)] = mref[e, pl.ds(k * L, L)] * w
            return cc

        lax.fori_loop(0, B // 4, ebody, 0)

    def chunk(ch, carry):
        @pl.when(ch > 0)
        def _():
            pltpu.make_async_copy(msg0, sagg.at[colstg.at[0]], sems0).wait()
            pltpu.make_async_copy(msg1, sagg.at[colstg.at[1]], sems1).wait()

        hb = base + ch * CH
        pltpu.sync_copy(row_hbm.at[pl.ds(hb, CH)], rowstg)
        pltpu.sync_copy(col_hbm.at[pl.ds(hb, CH)], colstg)
        pltpu.sync_copy(ew_hbm.at[pl.ds(hb, CH)], ewstg)
        pltpu.async_copy(x_hbm.at[rowstg.at[0]], msg0, semg0)

        def pair(t, cc):
            j0 = 2 * t
            j1 = j0 + 1
            pltpu.make_async_copy(x_hbm.at[rowstg.at[j0]], msg0, semg0).wait()

            @pl.when(t > 0)
            def _():
                pltpu.make_async_copy(
                    msg1, sagg.at[colstg.at[j1]], sems1).wait()

            pltpu.async_copy(x_hbm.at[rowstg.at[j1]], msg1, semg1)
            _scale(j0)
            _apply(msg0)
            pltpu.async_copy(msg0, sagg.at[colstg.at[j0]], sems0, add=True)
            pltpu.make_async_copy(x_hbm.at[rowstg.at[j1]], msg1, semg1).wait()
            _scale(j1)
            _apply(msg1)

            @pl.when(t < CH // 2 - 1)
            def _():
                pltpu.make_async_copy(
                    msg0, sagg.at[colstg.at[j0]], sems0).wait()
                pltpu.async_copy(x_hbm.at[rowstg.at[j0 + 2]], msg0, semg0)

            pltpu.async_copy(msg1, sagg.at[colstg.at[j1]], sems1, add=True)
            return cc

        lax.fori_loop(0, CH // 2, pair, 0)
        return carry

    lax.fori_loop(0, NCHUNK, chunk, 0)
    pltpu.make_async_copy(msg0, sagg.at[colstg.at[0]], sems0).wait()
    pltpu.make_async_copy(msg1, sagg.at[colstg.at[1]], sems1).wait()
    plsc.subcore_barrier()
    pltpu.sync_copy(sagg.at[pl.ds(s * SLICE, SLICE)],
                    pagg_hbm.at[c, pl.ds(s * SLICE, SLICE)])


BLK = 2000


def _fin_body(dis_ref, pagg_ref, x_ref, wt_ref, b_ref, out_ref):
    dis = dis_ref[...]
    agg = pagg_ref[0] + pagg_ref[1]
    y = dis * agg + (2.0 * dis * dis) * x_ref[...]
    out_ref[...] = (
        jnp.dot(y, wt_ref[...], preferred_element_type=jnp.float32)
        + b_ref[...])


def kernel(x, edge_index, edge_weight, W, bias):
    row = edge_index[0]
    col = edge_index[1]
    e = edge_weight.shape[0]
    pad = EPAD - e
    pad_idx = jnp.arange(pad, dtype=jnp.int32) % N
    rowp = jnp.concatenate([row, pad_idx]).reshape(ROWS, B)
    colp = jnp.concatenate([col, pad_idx]).reshape(ROWS, B)
    ewp = jnp.concatenate(
        [edge_weight, jnp.zeros((pad,), jnp.float32)]).reshape(ROWS, B)

    pdeg = _deg_kernel(colp, ewp)
    pagg, dis = _agg_kernel(rowp, colp, ewp, x, pdeg)

    out = pl.pallas_call(
        _fin_body,
        grid=(N // BLK,),
        in_specs=[
            pl.BlockSpec((BLK, 1), lambda i: (i, 0)),
            pl.BlockSpec((NC, BLK, C), lambda i: (0, i, 0)),
            pl.BlockSpec((BLK, C), lambda i: (i, 0)),
            pl.BlockSpec((C, C), lambda i: (0, 0)),
            pl.BlockSpec((1, C), lambda i: (0, 0)),
        ],
        out_specs=pl.BlockSpec((BLK, C), lambda i: (i, 0)),
        out_shape=jax.ShapeDtypeStruct((N, C), jnp.float32),
    )(dis.reshape(NPAD, 1), pagg, x, W.T, bias.reshape(1, C))
    return (out, out)


# speedup vs baseline: 29.0461x; 1.0065x over previous
import functools

import jax
import jax.numpy as jnp
from jax import lax
from jax.experimental import pallas as pl
from jax.experimental.pallas import tpu as pltpu
import jax.experimental.pallas.tpu_sc as plsc

N = 10000
C = 128
NC, NS, L = 2, 16, 16
NW = NC * NS
NPAD = 10240
SLICE = NPAD // NS
B = 128
ROWS_W = 80
ROWS = NW * ROWS_W
EPAD = ROWS * B
HROWS = ROWS_W // 2
NSL = N // NS

_mesh = plsc.VectorSubcoreMesh(
    core_axis_name="c", subcore_axis_name="s", num_cores=NC, num_subcores=NS)


@functools.partial(
    pl.kernel,
    out_type=jax.ShapeDtypeStruct((NC, NPAD), jnp.float32),
    mesh=_mesh,
    compiler_params=pltpu.CompilerParams(needs_layout_passes=False),
    scratch_types=[
        pltpu.VMEM((ROWS_W, B), jnp.int32),
        pltpu.VMEM((ROWS_W, B), jnp.float32),
        pltpu.VMEM((SLICE,), jnp.float32),
        pltpu.VMEM_SHARED((NPAD,), jnp.float32),
        pltpu.SemaphoreType.DMA,
    ],
)
def _deg_kernel(col_hbm, ew_hbm, pdeg_hbm, colv, ewv, zv, sdeg, sem):
    c = lax.axis_index("c")
    s = lax.axis_index("s")
    for k in range(SLICE // L):
        zv[pl.ds(k * L, L)] = jnp.zeros((L,), jnp.float32)
    pltpu.sync_copy(zv, sdeg.at[pl.ds(s * SLICE, SLICE)])
    plsc.subcore_barrier()
    base = (c * NS + s) * ROWS_W
    pltpu.sync_copy(col_hbm.at[pl.ds(base, ROWS_W)], colv)
    pltpu.sync_copy(ew_hbm.at[pl.ds(base, ROWS_W)], ewv)

    def step(j, carry):
        pltpu.async_copy(ewv.at[j], sdeg.at[colv.at[j]], sem, add=True)
        return carry

    lax.fori_loop(0, ROWS_W, step, 0)

    def drain(j, carry):
        pltpu.make_async_copy(ewv.at[0], sdeg.at[colv.at[0]], sem).wait()
        return carry

    lax.fori_loop(0, ROWS_W, drain, 0)
    plsc.subcore_barrier()
    pltpu.sync_copy(sdeg.at[pl.ds(s * SLICE, SLICE)],
                    pdeg_hbm.at[c, pl.ds(s * SLICE, SLICE)])


CH = 8
NCHUNK = ROWS_W // CH
GBYTES = B * C * 4


@functools.partial(
    pl.kernel,
    out_type=(jax.ShapeDtypeStruct((NC, NPAD, C), jnp.float32),
              jax.ShapeDtypeStruct((NPAD,), jnp.float32)),
    mesh=_mesh,
    compiler_params=pltpu.CompilerParams(needs_layout_passes=False),
    scratch_types=[
        pltpu.VMEM((CH, B), jnp.int32),
        pltpu.VMEM((CH, B), jnp.int32),
        pltpu.VMEM((CH, B), jnp.float32),
        pltpu.VMEM((SLICE,), jnp.float32),
        pltpu.VMEM((SLICE,), jnp.float32),
        pltpu.VMEM((NPAD,), jnp.float32),
        pltpu.VMEM((B,), jnp.float32),
        pltpu.VMEM((B, C), jnp.float32),
        pltpu.VMEM((B, C), jnp.float32),
        pltpu.VMEM_SHARED((NPAD, C), jnp.float32),
        pltpu.VMEM_SHARED((NPAD,), jnp.float32),
        pltpu.SemaphoreType.DMA,
        pltpu.SemaphoreType.DMA,
        pltpu.SemaphoreType.DMA,
        pltpu.SemaphoreType.DMA,
    ],
)
def _agg_kernel(row_hbm, col_hbm, ew_hbm, x_hbm, pdeg_hbm,
                pagg_hbm, dis_hbm,
                rowstg, colstg, ewstg, pa, pb, dis, scale, msg0, msg1,
                sagg, sdis, semg0, semg1, sems0, sems1):
    c = lax.axis_index("c")
    s = lax.axis_index("s")

    def zrow(j, carry):
        for k in range(C // L):
            msg0[j, pl.ds(k * L, L)] = jnp.zeros((L,), jnp.float32)
        return carry

    lax.fori_loop(0, B, zrow, 0)
    for m in range(SLICE // B):
        pltpu.sync_copy(msg0, sagg.at[pl.ds(s * SLICE + m * B, B)])

    off = s * SLICE
    pltpu.sync_copy(pdeg_hbm.at[0, pl.ds(off, SLICE)], pa)
    pltpu.sync_copy(pdeg_hbm.at[1, pl.ds(off, SLICE)], pb)

    def newt(k, carry):
        sl = pl.ds(k * L, L)
        d = pa[sl] + pb[sl] + 2.0
        y = jnp.full((L,), 0.0015, jnp.float32)
        for _ in range(26):
            y = y * (1.5 - 0.5 * d * y * y)
        dis[pl.ds(off + k * L, L)] = jnp.where(d > 0.0, y, 0.0)
        return carry

    lax.fori_loop(0, SLICE // L, newt, 0)
    pltpu.sync_copy(dis.at[pl.ds(off, SLICE)], sdis.at[pl.ds(off, SLICE)])
    plsc.subcore_barrier()
    pltpu.sync_copy(sdis, dis)

    @pl.when(jnp.logical_and(c == 0, s == 0))
    def _():
        pltpu.sync_copy(sdis, dis_hbm)

    base = (c * NS + s) * ROWS_W

    def _scale(j):
        for k in range(B // L):
            rv = rowstg[j, pl.ds(k * L, L)]
            dv = plsc.load_gather(dis, [rv])
            scale[pl.ds(k * L, L)] = ewstg[j, pl.ds(k * L, L)] * dv

    def _apply(mref):
        def ebody(t, cc):
            for u in range(4):
                e = t * 4 + u
                w = plsc.load_gather(scale, [jnp.zeros((L,), jnp.int32) + e])
                for k in range(C // L):
                    mref[e, pl.ds(k * L, L)] = mref[e, pl.ds(k * L, L)] * w
            return cc

        lax.fori_loop(0, B // 4, ebody, 0)

    def chunk(ch, carry):
        @pl.when(ch > 0)
        def _():
            pltpu.make_async_copy(msg0, sagg.at[colstg.at[0]], sems0).wait()
            pltpu.make_async_copy(msg1, sagg.at[colstg.at[1]], sems1).wait()

        hb = base + ch * CH
        pltpu.sync_copy(row_hbm.at[pl.ds(hb, CH)], rowstg)
        pltpu.sync_copy(col_hbm.at[pl.ds(hb, CH)], colstg)
        pltpu.sync_copy(ew_hbm.at[pl.ds(hb, CH)], ewstg)
        pltpu.async_copy(x_hbm.at[rowstg.at[0]], msg0, semg0)

        def pair(t, cc):
            j0 = 2 * t
            j1 = j0 + 1

            @pl.when(t > 0)
            def _():
                pltpu.make_async_copy(
                    msg1, sagg.at[colstg.at[j1]], sems1).wait()

            pltpu.async_copy(x_hbm.at[rowstg.at[j1]], msg1, semg1)
            _scale(j0)
            pltpu.make_async_copy(x_hbm.at[rowstg.at[j0]], msg0, semg0).wait()
            _apply(msg0)
            pltpu.async_copy(msg0, sagg.at[colstg.at[j0]], sems0, add=True)
            _scale(j1)
            pltpu.make_async_copy(x_hbm.at[rowstg.at[j1]], msg1, semg1).wait()
            _apply(msg1)

            @pl.when(t < CH // 2 - 1)
            def _():
                pltpu.make_async_copy(
                    msg0, sagg.at[colstg.at[j0]], sems0).wait()
                pltpu.async_copy(x_hbm.at[rowstg.at[j0 + 2]], msg0, semg0)

            pltpu.async_copy(msg1, sagg.at[colstg.at[j1]], sems1, add=True)
            return cc

        lax.fori_loop(0, CH // 2, pair, 0)
        return carry

    lax.fori_loop(0, NCHUNK, chunk, 0)
    pltpu.make_async_copy(msg0, sagg.at[colstg.at[0]], sems0).wait()
    pltpu.make_async_copy(msg1, sagg.at[colstg.at[1]], sems1).wait()
    plsc.subcore_barrier()
    pltpu.sync_copy(sagg.at[pl.ds(s * SLICE, SLICE)],
                    pagg_hbm.at[c, pl.ds(s * SLICE, SLICE)])


BLK = 2000


def _fin_body(dis_ref, pagg_ref, x_ref, wt_ref, b_ref, out_ref):
    dis = dis_ref[...]
    agg = pagg_ref[0] + pagg_ref[1]
    y = dis * agg + (2.0 * dis * dis) * x_ref[...]
    out_ref[...] = (
        jnp.dot(y, wt_ref[...], preferred_element_type=jnp.float32)
        + b_ref[...])


def kernel(x, edge_index, edge_weight, W, bias):
    row = edge_index[0]
    col = edge_index[1]
    e = edge_weight.shape[0]
    pad = EPAD - e
    pad_idx = jnp.arange(pad, dtype=jnp.int32) % N
    rowp = jnp.concatenate([row, pad_idx]).reshape(ROWS, B)
    colp = jnp.concatenate([col, pad_idx]).reshape(ROWS, B)
    ewp = jnp.concatenate(
        [edge_weight, jnp.zeros((pad,), jnp.float32)]).reshape(ROWS, B)

    pdeg = _deg_kernel(colp, ewp)
    pagg, dis = _agg_kernel(rowp, colp, ewp, x, pdeg)

    out = pl.pallas_call(
        _fin_body,
        grid=(N // BLK,),
        in_specs=[
            pl.BlockSpec((BLK, 1), lambda i: (i, 0)),
            pl.BlockSpec((NC, BLK, C), lambda i: (0, i, 0)),
            pl.BlockSpec((BLK, C), lambda i: (i, 0)),
            pl.BlockSpec((C, C), lambda i: (0, 0)),
            pl.BlockSpec((1, C), lambda i: (0, 0)),
        ],
        out_specs=pl.BlockSpec((BLK, C), lambda i: (i, 0)),
        out_shape=jax.ShapeDtypeStruct((N, C), jnp.float32),
    )(dis.reshape(NPAD, 1), pagg, x, W.T, bias.reshape(1, C))
    return (out, out)


# speedup vs baseline: 33.4961x; 1.1532x over previous
import functools

import jax
import jax.numpy as jnp
from jax import lax
from jax.experimental import pallas as pl
from jax.experimental.pallas import tpu as pltpu
import jax.experimental.pallas.tpu_sc as plsc

N = 10000
C = 128
NC, NS, L = 2, 16, 16
NW = NC * NS
NPAD = 10240
SLICE = NPAD // NS
B = 128
ROWS_W = 80
ROWS = NW * ROWS_W
EPAD = ROWS * B
HROWS = ROWS_W // 2
NSL = N // NS

_mesh = plsc.VectorSubcoreMesh(
    core_axis_name="c", subcore_axis_name="s", num_cores=NC, num_subcores=NS)


@functools.partial(
    pl.kernel,
    out_type=jax.ShapeDtypeStruct((NC, NPAD), jnp.float32),
    mesh=_mesh,
    compiler_params=pltpu.CompilerParams(needs_layout_passes=False),
    scratch_types=[
        pltpu.VMEM((ROWS_W, B), jnp.int32),
        pltpu.VMEM((ROWS_W, B), jnp.float32),
        pltpu.VMEM((SLICE,), jnp.float32),
        pltpu.VMEM_SHARED((NPAD,), jnp.float32),
        pltpu.SemaphoreType.DMA,
    ],
)
def _deg_kernel(col_hbm, ew_hbm, pdeg_hbm, colv, ewv, zv, sdeg, sem):
    c = lax.axis_index("c")
    s = lax.axis_index("s")
    for k in range(SLICE // L):
        zv[pl.ds(k * L, L)] = jnp.zeros((L,), jnp.float32)
    pltpu.sync_copy(zv, sdeg.at[pl.ds(s * SLICE, SLICE)])
    plsc.subcore_barrier()
    base = (c * NS + s) * ROWS_W
    pltpu.sync_copy(col_hbm.at[pl.ds(base, ROWS_W)], colv)
    pltpu.sync_copy(ew_hbm.at[pl.ds(base, ROWS_W)], ewv)

    def step(j, carry):
        pltpu.async_copy(ewv.at[j], sdeg.at[colv.at[j]], sem, add=True)
        return carry

    lax.fori_loop(0, ROWS_W, step, 0)

    def drain(j, carry):
        pltpu.make_async_copy(ewv.at[0], sdeg.at[colv.at[0]], sem).wait()
        return carry

    lax.fori_loop(0, ROWS_W, drain, 0)
    plsc.subcore_barrier()
    pltpu.sync_copy(sdeg.at[pl.ds(s * SLICE, SLICE)],
                    pdeg_hbm.at[c, pl.ds(s * SLICE, SLICE)])


CH = 8
NCHUNK = ROWS_W // CH
GBYTES = B * C * 4


@functools.partial(
    pl.kernel,
    out_type=(jax.ShapeDtypeStruct((NC, NPAD, C), jnp.float32),
              jax.ShapeDtypeStruct((NPAD,), jnp.float32)),
    mesh=_mesh,
    compiler_params=pltpu.CompilerParams(needs_layout_passes=False),
    scratch_types=[
        pltpu.VMEM((CH, B), jnp.int32),
        pltpu.VMEM((CH, B), jnp.int32),
        pltpu.VMEM((CH, B), jnp.float32),
        pltpu.VMEM((SLICE,), jnp.float32),
        pltpu.VMEM((SLICE,), jnp.float32),
        pltpu.VMEM((NPAD,), jnp.float32),
        pltpu.VMEM((B,), jnp.float32),
        pltpu.VMEM((B, C), jnp.float32),
        pltpu.VMEM((B, C), jnp.float32),
        pltpu.VMEM_SHARED((NPAD, C), jnp.float32),
        pltpu.VMEM_SHARED((NPAD,), jnp.float32),
        pltpu.SemaphoreType.DMA,
        pltpu.SemaphoreType.DMA,
        pltpu.SemaphoreType.DMA,
        pltpu.SemaphoreType.DMA,
    ],
)
def _agg_kernel(row_hbm, col_hbm, ew_hbm, x_hbm, pdeg_hbm,
                pagg_hbm, dis_hbm,
                rowstg, colstg, ewstg, pa, pb, dis, scale, msg0, msg1,
                sagg, sdis, semg0, semg1, sems0, sems1):
    c = lax.axis_index("c")
    s = lax.axis_index("s")

    def zrow(j, carry):
        for k in range(C // L):
            msg0[j, pl.ds(k * L, L)] = jnp.zeros((L,), jnp.float32)
        return carry

    lax.fori_loop(0, B, zrow, 0)
    for m in range(SLICE // B):
        pltpu.async_copy(msg0, sagg.at[pl.ds(s * SLICE + m * B, B)], semg0)
    for m in range(SLICE // B):
        pltpu.make_async_copy(
            msg0, sagg.at[pl.ds(s * SLICE, B)], semg0).wait()

    off = s * SLICE
    pltpu.sync_copy(pdeg_hbm.at[0, pl.ds(off, SLICE)], pa)
    pltpu.sync_copy(pdeg_hbm.at[1, pl.ds(off, SLICE)], pb)

    @plsc.parallel_loop(0, SLICE // L, unroll=2)
    def _newt(k):
        sl = pl.ds(k * L, L)
        d = pa[sl] + pb[sl] + 2.0
        y = jnp.where(
            d < 8.0, 0.35355339,
            jnp.where(d < 128.0, 0.08838835,
                      jnp.where(d < 2048.0, 0.02209709,
                                jnp.where(d < 32768.0, 0.00552427,
                                          0.00138107))))
        for _ in range(8):
            y = y * (1.5 - 0.5 * d * y * y)
        dis[pl.ds(off + k * L, L)] = jnp.where(d > 0.0, y, 0.0)
    pltpu.sync_copy(dis.at[pl.ds(off, SLICE)], sdis.at[pl.ds(off, SLICE)])
    plsc.subcore_barrier()
    pltpu.sync_copy(sdis, dis)

    @pl.when(jnp.logical_and(c == 0, s == 0))
    def _():
        pltpu.sync_copy(sdis, dis_hbm)

    base = (c * NS + s) * ROWS_W

    def _scale(j):
        rvs = [rowstg[j, pl.ds(k * L, L)] for k in range(B // L)]
        dvs = [plsc.load_gather(dis, [rv]) for rv in rvs]
        for k in range(B // L):
            scale[pl.ds(k * L, L)] = ewstg[j, pl.ds(k * L, L)] * dvs[k]

    def _apply(mref):
        @plsc.parallel_loop(0, B, unroll=8)
        def _(e):
            w = plsc.load_gather(scale, [jnp.zeros((L,), jnp.int32) + e])
            for k in range(C // L):
                mref[e, pl.ds(k * L, L)] = mref[e, pl.ds(k * L, L)] * w

    def chunk(ch, carry):
        @pl.when(ch > 0)
        def _():
            pltpu.make_async_copy(msg0, sagg.at[colstg.at[0]], sems0).wait()
            pltpu.make_async_copy(msg1, sagg.at[colstg.at[1]], sems1).wait()

        hb = base + ch * CH
        pltpu.sync_copy(row_hbm.at[pl.ds(hb, CH)], rowstg)
        pltpu.sync_copy(col_hbm.at[pl.ds(hb, CH)], colstg)
        pltpu.sync_copy(ew_hbm.at[pl.ds(hb, CH)], ewstg)
        pltpu.async_copy(x_hbm.at[rowstg.at[0]], msg0, semg0)

        def pair(t, cc):
            j0 = 2 * t
            j1 = j0 + 1

            @pl.when(t > 0)
            def _():
                pltpu.make_async_copy(
                    msg1, sagg.at[colstg.at[j1]], sems1).wait()

            pltpu.async_copy(x_hbm.at[rowstg.at[j1]], msg1, semg1)
            _scale(j0)
            pltpu.make_async_copy(x_hbm.at[rowstg.at[j0]], msg0, semg0).wait()
            _apply(msg0)
            pltpu.async_copy(msg0, sagg.at[colstg.at[j0]], sems0, add=True)
            _scale(j1)
            pltpu.make_async_copy(x_hbm.at[rowstg.at[j1]], msg1, semg1).wait()
            _apply(msg1)

            @pl.when(t < CH // 2 - 1)
            def _():
                pltpu.make_async_copy(
                    msg0, sagg.at[colstg.at[j0]], sems0).wait()
                pltpu.async_copy(x_hbm.at[rowstg.at[j0 + 2]], msg0, semg0)

            pltpu.async_copy(msg1, sagg.at[colstg.at[j1]], sems1, add=True)
            return cc

        lax.fori_loop(0, CH // 2, pair, 0)
        return carry

    lax.fori_loop(0, NCHUNK, chunk, 0)
    pltpu.make_async_copy(msg0, sagg.at[colstg.at[0]], sems0).wait()
    pltpu.make_async_copy(msg1, sagg.at[colstg.at[1]], sems1).wait()
    plsc.subcore_barrier()
    pltpu.sync_copy(sagg.at[pl.ds(s * SLICE, SLICE)],
                    pagg_hbm.at[c, pl.ds(s * SLICE, SLICE)])


BLK = 2000


def _fin_body(dis_ref, pagg_ref, x_ref, wt_ref, b_ref, out_ref):
    dis = dis_ref[...]
    agg = pagg_ref[0] + pagg_ref[1]
    y = dis * agg + (2.0 * dis * dis) * x_ref[...]
    out_ref[...] = (
        jnp.dot(y, wt_ref[...], preferred_element_type=jnp.float32)
        + b_ref[...])


def kernel(x, edge_index, edge_weight, W, bias):
    row = edge_index[0]
    col = edge_index[1]
    e = edge_weight.shape[0]
    pad = EPAD - e
    pad_idx = jnp.arange(pad, dtype=jnp.int32) % N
    rowp = jnp.concatenate([row, pad_idx]).reshape(ROWS, B)
    colp = jnp.concatenate([col, pad_idx]).reshape(ROWS, B)
    ewp = jnp.concatenate(
        [edge_weight, jnp.zeros((pad,), jnp.float32)]).reshape(ROWS, B)

    pdeg = _deg_kernel(colp, ewp)
    pagg, dis = _agg_kernel(rowp, colp, ewp, x, pdeg)

    out = pl.pallas_call(
        _fin_body,
        grid=(N // BLK,),
        in_specs=[
            pl.BlockSpec((BLK, 1), lambda i: (i, 0)),
            pl.BlockSpec((NC, BLK, C), lambda i: (0, i, 0)),
            pl.BlockSpec((BLK, C), lambda i: (i, 0)),
            pl.BlockSpec((C, C), lambda i: (0, 0)),
            pl.BlockSpec((1, C), lambda i: (0, 0)),
        ],
        out_specs=pl.BlockSpec((BLK, C), lambda i: (i, 0)),
        out_shape=jax.ShapeDtypeStruct((N, C), jnp.float32),
    )(dis.reshape(NPAD, 1), pagg, x, W.T, bias.reshape(1, C))
    return (out, out)
